# baseline (speedup 1.0000x reference)
import functools

import jax
import jax.numpy as jnp
from jax import lax
from jax.experimental import pallas as pl
from jax.experimental.pallas import tpu as pltpu
from jax.experimental.pallas import tpu_sc as plsc

_BATCH = 16384
_D = 64
_NC = 2
_NS = 16
_NW = _NC * _NS
_BPW = _BATCH // _NW
_CHUNK = 128
_NCHUNK = _BPW // _CHUNK
_L = 16
_GROUPS = _BPW // _L


def _rsqrt16(x):
    x = jnp.maximum(x, jnp.float32(1e-24))
    i = lax.bitcast_convert_type(x, jnp.int32)
    i = jnp.int32(0x5F3759DF) - lax.shift_right_logical(i, 1)
    y = lax.bitcast_convert_type(i, jnp.float32)
    for _ in range(3):
        y = y * (jnp.float32(1.5) - jnp.float32(0.5) * x * y * y)
    return y


_mesh = plsc.VectorSubcoreMesh(core_axis_name="c", subcore_axis_name="s")


@functools.partial(
    pl.kernel,
    out_type=jax.ShapeDtypeStruct((_BATCH,), jnp.float32),
    mesh=_mesh,
    compiler_params=pltpu.CompilerParams(
        needs_layout_passes=False, use_tc_tiling_on_sc=False),
    scratch_types=[
        pltpu.VMEM((_NCHUNK, _CHUNK), jnp.int32),
        pltpu.VMEM((_NCHUNK, _CHUNK), jnp.int32),
        pltpu.VMEM((_NCHUNK, _CHUNK), jnp.int32),
        pltpu.VMEM((_BPW, _D), jnp.float32),
        pltpu.VMEM((_BPW, _D), jnp.float32),
        pltpu.VMEM((_BPW, _D), jnp.float32),
        pltpu.VMEM((_BPW,), jnp.float32),
        pltpu.SemaphoreType.DMA,
    ],
)
def _transe_sc(h_hbm, r_hbm, t_hbm, ent_hbm, rel_hbm, out_hbm,
               hidx, ridx, tidx, hrows, rrows, trows, score, sem):
    wid = lax.axis_index("s") * _NC + lax.axis_index("c")
    base = wid * _BPW

    for c in range(_NCHUNK):
        off = base + c * _CHUNK
        pltpu.sync_copy(h_hbm.at[pl.ds(off, _CHUNK)], hidx.at[c])
        pltpu.sync_copy(r_hbm.at[pl.ds(off, _CHUNK)], ridx.at[c])
        pltpu.sync_copy(t_hbm.at[pl.ds(off, _CHUNK)], tidx.at[c])

    copies = []
    for c in range(_NCHUNK):
        sl = pl.ds(c * _CHUNK, _CHUNK)
        copies.append(pltpu.async_copy(ent_hbm.at[hidx.at[c]], hrows.at[sl], sem))
        copies.append(pltpu.async_copy(ent_hbm.at[tidx.at[c]], trows.at[sl], sem))
        copies.append(pltpu.async_copy(rel_hbm.at[ridx.at[c]], rrows.at[sl], sem))
    for cp in copies:
        cp.wait()

    lanes = lax.iota(jnp.int32, _L)
    cols = [jnp.full((_L,), j, jnp.int32) for j in range(_D)]

    def body(g, carry):
        row = g * _L + lanes
        acc_h = jnp.zeros((_L,), jnp.float32)
        acc_t = jnp.zeros((_L,), jnp.float32)
        for j in range(_D):
            hv = plsc.load_gather(hrows, [row, cols[j]])
            tv = plsc.load_gather(trows, [row, cols[j]])
            acc_h = acc_h + hv * hv
            acc_t = acc_t + tv * tv
        inv_h = _rsqrt16(acc_h)
        inv_t = _rsqrt16(acc_t)
        s = jnp.zeros((_L,), jnp.float32)
        for j in range(_D):
            hv = plsc.load_gather(hrows, [row, cols[j]])
            rv = plsc.load_gather(rrows, [row, cols[j]])
            tv = plsc.load_gather(trows, [row, cols[j]])
            s = s + jnp.abs(hv * inv_h + rv - tv * inv_t)
        score[pl.ds(pl.multiple_of(g * _L, _L), _L)] = -s
        return carry

    lax.fori_loop(0, _GROUPS, body, 0)
    pltpu.sync_copy(score, out_hbm.at[pl.ds(base, _BPW)])


def kernel(h, r, t, entity_emb, relation_emb):
    return _transe_sc(h.astype(jnp.int32), r.astype(jnp.int32),
                      t.astype(jnp.int32), entity_emb, relation_emb)
